# Initial kernel scaffold; baseline (speedup 1.0000x reference)
import jax
import jax.numpy as jnp
from jax.experimental import pallas as pl


def kernel(pred, target):
    raise NotImplementedError("write your pallas kernel here")


# speedup vs baseline: 5.0728x; 5.0728x over previous
import functools

import jax
import jax.numpy as jnp
from jax import lax
from jax.experimental import pallas as pl
from jax.experimental.pallas import tpu as pltpu
from jax.experimental.pallas import tpu_sc as plsc

N = 1_000_000
NC = 2
NT = 16
CH = 63_488
NP = NT * CH
W = 15_872
NW = CH // W
NV = W // 16
RADIX = 256
NPASS = 4

_mesh = plsc.VectorSubcoreMesh(
    core_axis_name="c", subcore_axis_name="s", num_cores=NC, num_subcores=NT
)


def _digit(vec, shift):
  return lax.shift_right_logical(vec, jnp.full((16,), shift, jnp.int32)) & 0xFF


def _sc_body(keys, ranks, ka, pa, kb, pb,
             grid_sh, vs_sh, keywin, paybuf, posbuf,
             gridvm, offs, hist):
  cid = lax.axis_index("c")
  t = lax.axis_index("s")
  lane = lax.iota(jnp.int32, 16)
  zeros16 = jnp.zeros((16,), jnp.int32)

  my_keys = keys.at[cid]
  my_rank = ranks.at[cid]
  bufs = [(ka.at[cid], pa.at[cid]), (kb.at[cid], pb.at[cid])]

  def zero_hist():
    def zbody(i, c):
      hist[pl.ds(i * 16, 16)] = zeros16
      return c

    lax.fori_loop(0, RADIX // 16, zbody, 0)

  def hist_update(d):
    occ, lastm = plsc.scan_count(d)
    cur = plsc.load_gather(hist, [d])
    plsc.store_scatter(hist, [d], cur + occ, mask=lastm)

  def publish_hist():
    pltpu.sync_copy(hist, grid_sh.at[pl.ds(t * RADIX, RADIX)])

  zero_hist()

  def h0win(w, c):
    pltpu.sync_copy(my_keys.at[pl.ds(t * CH + w * W, W)], keywin)

    def hbody(i, c2):
      hist_update(_digit(keywin[pl.ds(i * 16, 16)], 0))
      return c2

    lax.fori_loop(0, NV, hbody, 0)
    return c

  lax.fori_loop(0, NW, h0win, 0)
  publish_hist()
  plsc.subcore_barrier()

  def compute_offs():
    def obody(c, carry):
      def racc(tp, tb):
        tot, below = tb
        row = gridvm[pl.ds(tp * RADIX + c * 16, 16)]
        return tot + row, below + jnp.where(tp < t, row, jnp.zeros_like(row))

      tot, below = lax.fori_loop(0, NT, racc, (zeros16, zeros16))
      cs = plsc.cumsum(tot)
      offs[pl.ds(c * 16, 16)] = carry + (cs - tot) + below
      return carry + jnp.sum(tot)

    lax.fori_loop(0, RADIX // 16, obody, jnp.int32(0))

  def one_pass(p):
    shift = 8 * p
    first = p == 0
    is_last = p == NPASS - 1
    src_k = my_keys if first else bufs[(p + 1) % 2][0]
    src_p = None if first else bufs[(p + 1) % 2][1]
    dst_k, dst_p = bufs[p % 2]

    pltpu.sync_copy(grid_sh, gridvm)
    compute_offs()

    if not is_last:
      def kwin(w, c):
        off = t * CH + w * W
        pltpu.sync_copy(src_k.at[pl.ds(off, W)], keywin)

        def pbody(i, c2):
          d = _digit(keywin[pl.ds(i * 16, 16)], shift)
          occ, lastm = plsc.scan_count(d)
          cur = plsc.load_gather(offs, [d])
          posbuf[pl.ds(i * 16, 16)] = cur + occ - 1
          plsc.store_scatter(offs, [d], cur + occ, mask=lastm)
          return c2

        lax.fori_loop(0, NV, pbody, 0)
        pltpu.sync_copy(keywin, vs_sh.at[posbuf])
        return c

      lax.fori_loop(0, NW, kwin, 0)
      plsc.subcore_barrier()

      zero_hist()

      def ckwin(w, c):
        off = t * CH + w * W
        pltpu.sync_copy(vs_sh.at[pl.ds(off, W)], keywin)

        def cbody(i, c2):
          hist_update(_digit(keywin[pl.ds(i * 16, 16)], shift + 8))
          return c2

        lax.fori_loop(0, NV, cbody, 0)
        pltpu.sync_copy(keywin, dst_k.at[pl.ds(off, W)])
        return c

      lax.fori_loop(0, NW, ckwin, 0)
      publish_hist()
      plsc.subcore_barrier()

    compute_offs()

    def pwin(w, c):
      off = t * CH + w * W
      pltpu.sync_copy(src_k.at[pl.ds(off, W)], keywin)
      if first:
        pass
      else:
        pltpu.sync_copy(src_p.at[pl.ds(off, W)], paybuf)

      def pbody(i, c2):
        d = _digit(keywin[pl.ds(i * 16, 16)], shift)
        occ, lastm = plsc.scan_count(d)
        cur = plsc.load_gather(offs, [d])
        posbuf[pl.ds(i * 16, 16)] = cur + occ - 1
        plsc.store_scatter(offs, [d], cur + occ, mask=lastm)
        if first:
          paybuf[pl.ds(i * 16, 16)] = off + i * 16 + lane
        return c2

      lax.fori_loop(0, NV, pbody, 0)
      if is_last:
        pltpu.sync_copy(posbuf, vs_sh.at[paybuf])
      else:
        pltpu.sync_copy(paybuf, vs_sh.at[posbuf])
      return c

    lax.fori_loop(0, NW, pwin, 0)
    plsc.subcore_barrier()

    dst = my_rank if is_last else dst_p
    pltpu.sync_copy(vs_sh.at[pl.ds(t * CH, CH)], dst.at[pl.ds(t * CH, CH)])
    plsc.subcore_barrier()

  for p in range(NPASS):
    one_pass(p)


_buf = jax.ShapeDtypeStruct((NC, NP), jnp.int32)
_sc_rank = functools.partial(
    pl.kernel,
    out_type=(_buf, _buf, _buf, _buf, _buf),
    mesh=_mesh,
    scratch_types=[
        pltpu.VMEM_SHARED((NT * RADIX,), jnp.int32),
        pltpu.VMEM_SHARED((NP,), jnp.int32),
        pltpu.VMEM((W,), jnp.int32),
        pltpu.VMEM((W,), jnp.int32),
        pltpu.VMEM((W,), jnp.int32),
        pltpu.VMEM((NT * RADIX,), jnp.int32),
        pltpu.VMEM((RADIX,), jnp.int32),
        pltpu.VMEM((RADIX,), jnp.int32),
    ],
    compiler_params=pltpu.CompilerParams(needs_layout_passes=False),
)(_sc_body)


ROWS = NP // 128
BLK = 128
GRID = ROWS // BLK
MEAN = (N - 1) / 2.0


def _dot_body(rp_ref, rt_ref, acc_ref):
  i = pl.program_id(0)
  rp = rp_ref[...].astype(jnp.float32) - MEAN
  rt = rt_ref[...].astype(jnp.float32) - MEAN
  r = lax.broadcasted_iota(jnp.int32, (BLK, 128), 0)
  c = lax.broadcasted_iota(jnp.int32, (BLK, 128), 1)
  gidx = (i * BLK + r) * 128 + c
  s = jnp.sum(jnp.where(gidx < N, rp * rt, 0.0))

  @pl.when(i == 0)
  def _():
    acc_ref[0, 0] = 0.0

  acc_ref[0, 0] += s


_dot = pl.pallas_call(
    _dot_body,
    grid=(GRID,),
    in_specs=[
        pl.BlockSpec((BLK, 128), lambda i: (i, 0)),
        pl.BlockSpec((BLK, 128), lambda i: (i, 0)),
    ],
    out_specs=pl.BlockSpec(memory_space=pltpu.SMEM),
    out_shape=jax.ShapeDtypeStruct((1, 1), jnp.float32),
)


def kernel(pred, target):
  bp = lax.bitcast_convert_type(pred, jnp.int32)
  bt = lax.bitcast_convert_type(target, jnp.int32)
  sign = jnp.int32(-2147483648)
  kp = jnp.where(bp >= 0, bp ^ sign, ~bp)
  kt = jnp.where(bt >= 0, bt ^ sign, ~bt)
  pad = jnp.full((NP - N,), -1, jnp.int32)
  keys = jnp.stack([jnp.concatenate([kp, pad]), jnp.concatenate([kt, pad])])
  ranks, _, _, _, _ = _sc_rank(keys)
  rp2d = ranks[0].reshape(ROWS, 128)
  rt2d = ranks[1].reshape(ROWS, 128)
  s = _dot(rp2d, rt2d)[0, 0]
  denom = float(N) * (float(N) * (N + 1) / 12.0)
  return s * jnp.float32(1.0 / denom)


# speedup vs baseline: 7.0365x; 1.3871x over previous
import functools

import jax
import jax.numpy as jnp
from jax import lax
from jax.experimental import pallas as pl
from jax.experimental.pallas import tpu as pltpu
from jax.experimental.pallas import tpu_sc as plsc

N = 1_000_000
NC = 2
NT = 16
CH = 63_488
NP = NT * CH
W = 15_872
NW = CH // W
NV = W // 16
RADIX = 256
NPASS = 4

_mesh = plsc.VectorSubcoreMesh(
    core_axis_name="c", subcore_axis_name="s", num_cores=NC, num_subcores=NT
)


def _digit(vec, shift):
  return lax.shift_right_logical(vec, jnp.full((16,), shift, jnp.int32)) & 0xFF


def _sc_body(keys, iota_in, ranks, ka, pa, kb, pb, posdump,
             grid_sh, vs_sh, keywin, paybuf, posbuf,
             gridvm, offs, hist):
  cid = lax.axis_index("c")
  t = lax.axis_index("s")
  lane = lax.iota(jnp.int32, 16)
  zeros16 = jnp.zeros((16,), jnp.int32)
  ones16 = jnp.full((16,), 1, jnp.int32)

  my_keys = keys.at[cid]
  my_rank = ranks.at[cid]
  my_pd = posdump.at[cid]
  bufs = [(ka.at[cid], pa.at[cid]), (kb.at[cid], pb.at[cid])]

  def zero_hist():
    def zbody(i, c):
      hist[pl.ds(i * 16, 16)] = zeros16
      return c

    lax.fori_loop(0, RADIX // 16, zbody, 0)

  def hist_update(d):
    plsc.addupdate_scatter(hist, [d], ones16)

  def publish_hist():
    pltpu.sync_copy(hist, grid_sh.at[pl.ds(t * RADIX, RADIX)])

  zero_hist()

  def h0win(w, c):
    pltpu.sync_copy(my_keys.at[pl.ds(t * CH + w * W, W)], keywin)

    def hbody(i, c2):
      hist_update(_digit(keywin[pl.ds(i * 16, 16)], 0))
      return c2

    lax.fori_loop(0, NV, hbody, 0)
    return c

  lax.fori_loop(0, NW, h0win, 0)
  publish_hist()
  plsc.subcore_barrier()

  def compute_offs():
    def obody(c, carry):
      def racc(tp, tb):
        tot, below = tb
        row = gridvm[pl.ds(tp * RADIX + c * 16, 16)]
        return tot + row, below + jnp.where(tp < t, row, jnp.zeros_like(row))

      tot, below = lax.fori_loop(0, NT, racc, (zeros16, zeros16))
      cs = plsc.cumsum(tot)
      offs[pl.ds(c * 16, 16)] = carry + (cs - tot) + below
      return carry + jnp.sum(tot)

    lax.fori_loop(0, RADIX // 16, obody, jnp.int32(0))

  def one_pass(p):
    shift = 8 * p
    first = p == 0
    is_last = p == NPASS - 1
    src_k = my_keys if first else bufs[(p + 1) % 2][0]
    src_p = None if first else bufs[(p + 1) % 2][1]
    dst_k, dst_p = bufs[p % 2]

    pltpu.sync_copy(grid_sh, gridvm)
    compute_offs()

    if not is_last:
      def kwin(w, c):
        off = t * CH + w * W
        pltpu.sync_copy(src_k.at[pl.ds(off, W)], keywin)

        def pbody(i, c2):
          d = _digit(keywin[pl.ds(i * 16, 16)], shift)
          occ, lastm = plsc.scan_count(d)
          cur = plsc.load_gather(offs, [d])
          posbuf[pl.ds(i * 16, 16)] = cur + occ - 1
          plsc.store_scatter(offs, [d], cur + occ, mask=lastm)
          return c2

        lax.fori_loop(0, NV, pbody, 0)
        pltpu.sync_copy(keywin, vs_sh.at[posbuf])
        pltpu.sync_copy(posbuf, my_pd.at[pl.ds(off, W)])
        return c

      lax.fori_loop(0, NW, kwin, 0)
      plsc.subcore_barrier()

      zero_hist()

      def ckwin(w, c):
        off = t * CH + w * W
        pltpu.sync_copy(vs_sh.at[pl.ds(off, W)], keywin)

        def cbody(i, c2):
          hist_update(_digit(keywin[pl.ds(i * 16, 16)], shift + 8))
          return c2

        lax.fori_loop(0, NV, cbody, 0)
        pltpu.sync_copy(keywin, dst_k.at[pl.ds(off, W)])
        return c

      lax.fori_loop(0, NW, ckwin, 0)
      publish_hist()
      plsc.subcore_barrier()

      src_pay = iota_in if first else src_p

      def pwin(w, c):
        off = t * CH + w * W
        pltpu.sync_copy(my_pd.at[pl.ds(off, W)], posbuf)
        pltpu.sync_copy(src_pay.at[pl.ds(off, W)], paybuf)
        pltpu.sync_copy(paybuf, vs_sh.at[posbuf])
        return c

      lax.fori_loop(0, NW, pwin, 0)
      plsc.subcore_barrier()
    else:
      def pwin(w, c):
        off = t * CH + w * W
        pltpu.sync_copy(src_k.at[pl.ds(off, W)], keywin)
        pltpu.sync_copy(src_p.at[pl.ds(off, W)], paybuf)

        def pbody(i, c2):
          d = _digit(keywin[pl.ds(i * 16, 16)], shift)
          occ, lastm = plsc.scan_count(d)
          cur = plsc.load_gather(offs, [d])
          posbuf[pl.ds(i * 16, 16)] = cur + occ - 1
          plsc.store_scatter(offs, [d], cur + occ, mask=lastm)
          return c2

        lax.fori_loop(0, NV, pbody, 0)
        pltpu.sync_copy(posbuf, vs_sh.at[paybuf])
        return c

      lax.fori_loop(0, NW, pwin, 0)
      plsc.subcore_barrier()

    dst = my_rank if is_last else dst_p
    pltpu.sync_copy(vs_sh.at[pl.ds(t * CH, CH)], dst.at[pl.ds(t * CH, CH)])
    plsc.subcore_barrier()

  for p in range(NPASS):
    one_pass(p)


_buf = jax.ShapeDtypeStruct((NC, NP), jnp.int32)
_sc_rank = functools.partial(
    pl.kernel,
    out_type=(_buf, _buf, _buf, _buf, _buf, _buf),
    mesh=_mesh,
    scratch_types=[
        pltpu.VMEM_SHARED((NT * RADIX,), jnp.int32),
        pltpu.VMEM_SHARED((NP,), jnp.int32),
        pltpu.VMEM((W,), jnp.int32),
        pltpu.VMEM((W,), jnp.int32),
        pltpu.VMEM((W,), jnp.int32),
        pltpu.VMEM((NT * RADIX,), jnp.int32),
        pltpu.VMEM((RADIX,), jnp.int32),
        pltpu.VMEM((RADIX,), jnp.int32),
    ],
    compiler_params=pltpu.CompilerParams(needs_layout_passes=False),
)(_sc_body)


ROWS = NP // 128
BLK = 128
GRID = ROWS // BLK
MEAN = (N - 1) / 2.0


def _dot_body(rp_ref, rt_ref, acc_ref):
  i = pl.program_id(0)
  rp = rp_ref[...].astype(jnp.float32) - MEAN
  rt = rt_ref[...].astype(jnp.float32) - MEAN
  r = lax.broadcasted_iota(jnp.int32, (BLK, 128), 0)
  c = lax.broadcasted_iota(jnp.int32, (BLK, 128), 1)
  gidx = (i * BLK + r) * 128 + c
  s = jnp.sum(jnp.where(gidx < N, rp * rt, 0.0))

  @pl.when(i == 0)
  def _():
    acc_ref[0, 0] = 0.0

  acc_ref[0, 0] += s


_dot = pl.pallas_call(
    _dot_body,
    grid=(GRID,),
    in_specs=[
        pl.BlockSpec((BLK, 128), lambda i: (i, 0)),
        pl.BlockSpec((BLK, 128), lambda i: (i, 0)),
    ],
    out_specs=pl.BlockSpec(memory_space=pltpu.SMEM),
    out_shape=jax.ShapeDtypeStruct((1, 1), jnp.float32),
)


def kernel(pred, target):
  bp = lax.bitcast_convert_type(pred, jnp.int32)
  bt = lax.bitcast_convert_type(target, jnp.int32)
  sign = jnp.int32(-2147483648)
  kp = jnp.where(bp >= 0, bp ^ sign, ~bp)
  kt = jnp.where(bt >= 0, bt ^ sign, ~bt)
  pad = jnp.full((NP - N,), -1, jnp.int32)
  keys = jnp.stack([jnp.concatenate([kp, pad]), jnp.concatenate([kt, pad])])
  iota = jnp.arange(NP, dtype=jnp.int32)
  ranks = _sc_rank(keys, iota)[0]
  rp2d = ranks[0].reshape(ROWS, 128)
  rt2d = ranks[1].reshape(ROWS, 128)
  s = _dot(rp2d, rt2d)[0, 0]
  denom = float(N) * (float(N) * (N + 1) / 12.0)
  return s * jnp.float32(1.0 / denom)


# speedup vs baseline: 8.5271x; 1.2118x over previous
import functools

import jax
import jax.numpy as jnp
from jax import lax
from jax.experimental import pallas as pl
from jax.experimental.pallas import tpu as pltpu
from jax.experimental.pallas import tpu_sc as plsc

N = 1_000_000
NC = 2
NT = 16
CH = 63_488
NP = NT * CH
W = 15_872
NW = CH // W
NV = W // 16
RADIX = 256
NPASS = 4

_mesh = plsc.VectorSubcoreMesh(
    core_axis_name="c", subcore_axis_name="s", num_cores=NC, num_subcores=NT
)


def _digit(vec, shift):
  return lax.shift_right_logical(vec, jnp.full((16,), shift, jnp.int32)) & 0xFF


def _sc_body(keys, iota_in, ranks, ka, pa, kb, pb, posdump,
             grid_sh, vs_sh, keywin, paybuf, posbuf,
             gridvm, offs, hist):
  cid = lax.axis_index("c")
  t = lax.axis_index("s")
  lane = lax.iota(jnp.int32, 16)
  zeros16 = jnp.zeros((16,), jnp.int32)
  ones16 = jnp.full((16,), 1, jnp.int32)

  my_keys = keys.at[cid]
  my_rank = ranks.at[cid]
  my_pd = posdump.at[cid]
  bufs = [(ka.at[cid], pa.at[cid]), (kb.at[cid], pb.at[cid])]

  def zero_hist():
    def zbody(i, c):
      hist[pl.ds(i * 16, 16)] = zeros16
      return c

    lax.fori_loop(0, RADIX // 16, zbody, 0)

  def hist_update(d):
    plsc.addupdate_scatter(hist, [d], ones16)

  def publish_hist():
    pltpu.sync_copy(hist, grid_sh.at[pl.ds(t * RADIX, RADIX)])

  zero_hist()

  def h0win(w, c):
    pltpu.sync_copy(my_keys.at[pl.ds(t * CH + w * W, W)], keywin)

    def hbody(i, c2):
      d0 = _digit(keywin[pl.ds(i * 32, 16)], 0)
      d1 = _digit(keywin[pl.ds(i * 32 + 16, 16)], 0)
      hist_update(d0)
      hist_update(d1)
      return c2

    lax.fori_loop(0, NV // 2, hbody, 0)
    return c

  lax.fori_loop(0, NW, h0win, 0)
  publish_hist()
  plsc.subcore_barrier()

  def compute_offs():
    def obody(c, carry):
      def racc(tp, tb):
        tot, below = tb
        row = gridvm[pl.ds(tp * RADIX + c * 16, 16)]
        return tot + row, below + jnp.where(tp < t, row, jnp.zeros_like(row))

      tot, below = lax.fori_loop(0, NT, racc, (zeros16, zeros16))
      cs = plsc.cumsum(tot)
      offs[pl.ds(c * 16, 16)] = carry + (cs - tot) + below
      return carry + jnp.sum(tot)

    lax.fori_loop(0, RADIX // 16, obody, jnp.int32(0))

  def one_pass(p):
    shift = 8 * p
    first = p == 0
    is_last = p == NPASS - 1
    src_k = my_keys if first else bufs[(p + 1) % 2][0]
    src_p = None if first else bufs[(p + 1) % 2][1]
    dst_k, dst_p = bufs[p % 2]

    pltpu.sync_copy(grid_sh, gridvm)
    compute_offs()

    if not is_last:
      def kwin(w, c):
        off = t * CH + w * W
        pltpu.sync_copy(src_k.at[pl.ds(off, W)], keywin)

        def pbody(i, c2):
          d0 = _digit(keywin[pl.ds(i * 32, 16)], shift)
          d1 = _digit(keywin[pl.ds(i * 32 + 16, 16)], shift)
          occ0, l0 = plsc.scan_count(d0)
          occ1, l1 = plsc.scan_count(d1)
          cur0 = plsc.load_gather(offs, [d0])
          posbuf[pl.ds(i * 32, 16)] = cur0 + occ0 - 1
          plsc.store_scatter(offs, [d0], cur0 + occ0, mask=l0)
          cur1 = plsc.load_gather(offs, [d1])
          posbuf[pl.ds(i * 32 + 16, 16)] = cur1 + occ1 - 1
          plsc.store_scatter(offs, [d1], cur1 + occ1, mask=l1)
          return c2

        lax.fori_loop(0, NV // 2, pbody, 0)
        pltpu.sync_copy(keywin, vs_sh.at[posbuf])
        pltpu.sync_copy(posbuf, my_pd.at[pl.ds(off, W)])
        return c

      lax.fori_loop(0, NW, kwin, 0)
      plsc.subcore_barrier()

      zero_hist()

      def ckwin(w, c):
        off = t * CH + w * W
        pltpu.sync_copy(vs_sh.at[pl.ds(off, W)], keywin)

        def cbody(i, c2):
          d0 = _digit(keywin[pl.ds(i * 32, 16)], shift + 8)
          d1 = _digit(keywin[pl.ds(i * 32 + 16, 16)], shift + 8)
          hist_update(d0)
          hist_update(d1)
          return c2

        lax.fori_loop(0, NV // 2, cbody, 0)
        pltpu.sync_copy(keywin, dst_k.at[pl.ds(off, W)])
        return c

      lax.fori_loop(0, NW, ckwin, 0)
      publish_hist()
      plsc.subcore_barrier()

      src_pay = iota_in if first else src_p

      def pwin(w, c):
        off = t * CH + w * W
        pltpu.sync_copy(my_pd.at[pl.ds(off, W)], posbuf)
        pltpu.sync_copy(src_pay.at[pl.ds(off, W)], paybuf)
        pltpu.sync_copy(paybuf, vs_sh.at[posbuf])
        return c

      lax.fori_loop(0, NW, pwin, 0)
      plsc.subcore_barrier()
    else:
      def pwin(w, c):
        off = t * CH + w * W
        pltpu.sync_copy(src_k.at[pl.ds(off, W)], keywin)
        pltpu.sync_copy(src_p.at[pl.ds(off, W)], paybuf)

        def pbody(i, c2):
          d0 = _digit(keywin[pl.ds(i * 32, 16)], shift)
          d1 = _digit(keywin[pl.ds(i * 32 + 16, 16)], shift)
          occ0, l0 = plsc.scan_count(d0)
          occ1, l1 = plsc.scan_count(d1)
          cur0 = plsc.load_gather(offs, [d0])
          posbuf[pl.ds(i * 32, 16)] = cur0 + occ0 - 1
          plsc.store_scatter(offs, [d0], cur0 + occ0, mask=l0)
          cur1 = plsc.load_gather(offs, [d1])
          posbuf[pl.ds(i * 32 + 16, 16)] = cur1 + occ1 - 1
          plsc.store_scatter(offs, [d1], cur1 + occ1, mask=l1)
          return c2

        lax.fori_loop(0, NV // 2, pbody, 0)
        pltpu.sync_copy(posbuf, vs_sh.at[paybuf])
        return c

      lax.fori_loop(0, NW, pwin, 0)
      plsc.subcore_barrier()

    dst = my_rank if is_last else dst_p
    pltpu.sync_copy(vs_sh.at[pl.ds(t * CH, CH)], dst.at[pl.ds(t * CH, CH)])
    plsc.subcore_barrier()

  for p in range(NPASS):
    one_pass(p)


_buf = jax.ShapeDtypeStruct((NC, NP), jnp.int32)
_sc_rank = functools.partial(
    pl.kernel,
    out_type=(_buf, _buf, _buf, _buf, _buf, _buf),
    mesh=_mesh,
    scratch_types=[
        pltpu.VMEM_SHARED((NT * RADIX,), jnp.int32),
        pltpu.VMEM_SHARED((NP,), jnp.int32),
        pltpu.VMEM((W,), jnp.int32),
        pltpu.VMEM((W,), jnp.int32),
        pltpu.VMEM((W,), jnp.int32),
        pltpu.VMEM((NT * RADIX,), jnp.int32),
        pltpu.VMEM((RADIX,), jnp.int32),
        pltpu.VMEM((RADIX,), jnp.int32),
    ],
    compiler_params=pltpu.CompilerParams(needs_layout_passes=False),
)(_sc_body)


ROWS = NP // 128
BLK = 128
GRID = ROWS // BLK
MEAN = (N - 1) / 2.0


def _dot_body(rp_ref, rt_ref, acc_ref):
  i = pl.program_id(0)
  rp = rp_ref[...].astype(jnp.float32) - MEAN
  rt = rt_ref[...].astype(jnp.float32) - MEAN
  r = lax.broadcasted_iota(jnp.int32, (BLK, 128), 0)
  c = lax.broadcasted_iota(jnp.int32, (BLK, 128), 1)
  gidx = (i * BLK + r) * 128 + c
  s = jnp.sum(jnp.where(gidx < N, rp * rt, 0.0))

  @pl.when(i == 0)
  def _():
    acc_ref[0, 0] = 0.0

  acc_ref[0, 0] += s


_dot = pl.pallas_call(
    _dot_body,
    grid=(GRID,),
    in_specs=[
        pl.BlockSpec((BLK, 128), lambda i: (i, 0)),
        pl.BlockSpec((BLK, 128), lambda i: (i, 0)),
    ],
    out_specs=pl.BlockSpec(memory_space=pltpu.SMEM),
    out_shape=jax.ShapeDtypeStruct((1, 1), jnp.float32),
)


def kernel(pred, target):
  bp = lax.bitcast_convert_type(pred, jnp.int32)
  bt = lax.bitcast_convert_type(target, jnp.int32)
  sign = jnp.int32(-2147483648)
  kp = jnp.where(bp >= 0, bp ^ sign, ~bp)
  kt = jnp.where(bt >= 0, bt ^ sign, ~bt)
  pad = jnp.full((NP - N,), -1, jnp.int32)
  keys = jnp.stack([jnp.concatenate([kp, pad]), jnp.concatenate([kt, pad])])
  iota = jnp.arange(NP, dtype=jnp.int32)
  ranks = _sc_rank(keys, iota)[0]
  rp2d = ranks[0].reshape(ROWS, 128)
  rt2d = ranks[1].reshape(ROWS, 128)
  s = _dot(rp2d, rt2d)[0, 0]
  denom = float(N) * (float(N) * (N + 1) / 12.0)
  return s * jnp.float32(1.0 / denom)


# speedup vs baseline: 8.6875x; 1.0188x over previous
import functools

import jax
import jax.numpy as jnp
from jax import lax
from jax.experimental import pallas as pl
from jax.experimental.pallas import tpu as pltpu
from jax.experimental.pallas import tpu_sc as plsc

N = 1_000_000
NC = 2
NT = 16
CH = 63_488
NP = NT * CH
W = 15_872
NW = CH // W
NV = W // 16
RADIX = 256
NPASS = 4

_mesh = plsc.VectorSubcoreMesh(
    core_axis_name="c", subcore_axis_name="s", num_cores=NC, num_subcores=NT
)


def _digit(vec, shift):
  return lax.shift_right_logical(vec, jnp.full((16,), shift, jnp.int32)) & 0xFF


UNROLL = 4


def _rank_vregs(keywin, posbuf, offs, shift):
  def body(i, c):
    base = i * (16 * UNROLL)
    ds = [_digit(keywin[pl.ds(base + 16 * u, 16)], shift) for u in range(UNROLL)]
    sc = [plsc.scan_count(d) for d in ds]
    for u in range(UNROLL):
      occ, lastm = sc[u]
      cur = plsc.load_gather(offs, [ds[u]])
      posbuf[pl.ds(base + 16 * u, 16)] = cur + occ - 1
      plsc.store_scatter(offs, [ds[u]], cur + occ, mask=lastm)
    return c

  return body


def _sc_body(keys, iota_in, ranks, ka, pa, kb, pb, posdump,
             grid_sh, vs_sh, keywin, paybuf, posbuf,
             gridvm, offs, hist, sem1, sem2):
  cid = lax.axis_index("c")
  t = lax.axis_index("s")
  lane = lax.iota(jnp.int32, 16)
  zeros16 = jnp.zeros((16,), jnp.int32)
  ones16 = jnp.full((16,), 1, jnp.int32)

  my_keys = keys.at[cid]
  my_rank = ranks.at[cid]
  my_pd = posdump.at[cid]
  bufs = [(ka.at[cid], pa.at[cid]), (kb.at[cid], pb.at[cid])]

  def zero_hist():
    def zbody(i, c):
      hist[pl.ds(i * 16, 16)] = zeros16
      return c

    lax.fori_loop(0, RADIX // 16, zbody, 0)

  def hist_update(d):
    plsc.addupdate_scatter(hist, [d], ones16)

  def publish_hist():
    pltpu.sync_copy(hist, grid_sh.at[pl.ds(t * RADIX, RADIX)])

  zero_hist()

  def h0win(w, c):
    pltpu.sync_copy(my_keys.at[pl.ds(t * CH + w * W, W)], keywin)

    def hbody(i, c2):
      for u in range(UNROLL):
        hist_update(_digit(keywin[pl.ds(i * 16 * UNROLL + 16 * u, 16)], 0))
      return c2

    lax.fori_loop(0, NV // UNROLL, hbody, 0)
    return c

  lax.fori_loop(0, NW, h0win, 0)
  publish_hist()
  plsc.subcore_barrier()

  def compute_offs():
    def obody(c, carry):
      def racc(tp, tb):
        tot, below = tb
        row = gridvm[pl.ds(tp * RADIX + c * 16, 16)]
        return tot + row, below + jnp.where(tp < t, row, jnp.zeros_like(row))

      tot, below = lax.fori_loop(0, NT, racc, (zeros16, zeros16))
      cs = plsc.cumsum(tot)
      offs[pl.ds(c * 16, 16)] = carry + (cs - tot) + below
      return carry + jnp.sum(tot)

    lax.fori_loop(0, RADIX // 16, obody, jnp.int32(0))

  def one_pass(p):
    shift = 8 * p
    first = p == 0
    is_last = p == NPASS - 1
    src_k = my_keys if first else bufs[(p + 1) % 2][0]
    src_p = None if first else bufs[(p + 1) % 2][1]
    dst_k, dst_p = bufs[p % 2]

    pltpu.sync_copy(grid_sh, gridvm)
    compute_offs()

    if not is_last:
      def kwin(w, c):
        off = t * CH + w * W
        pltpu.sync_copy(src_k.at[pl.ds(off, W)], keywin)

        lax.fori_loop(0, NV // UNROLL, _rank_vregs(keywin, posbuf, offs, shift), 0)
        cp1 = pltpu.async_copy(keywin, vs_sh.at[posbuf], sem1)
        cp2 = pltpu.async_copy(posbuf, my_pd.at[pl.ds(off, W)], sem2)
        cp1.wait()
        cp2.wait()
        return c

      lax.fori_loop(0, NW, kwin, 0)
      plsc.subcore_barrier()

      zero_hist()

      def ckwin(w, c):
        off = t * CH + w * W
        pltpu.sync_copy(vs_sh.at[pl.ds(off, W)], keywin)

        def cbody(i, c2):
          for u in range(UNROLL):
            hist_update(
                _digit(keywin[pl.ds(i * 16 * UNROLL + 16 * u, 16)], shift + 8))
          return c2

        lax.fori_loop(0, NV // UNROLL, cbody, 0)
        pltpu.sync_copy(keywin, dst_k.at[pl.ds(off, W)])
        return c

      lax.fori_loop(0, NW, ckwin, 0)
      publish_hist()
      plsc.subcore_barrier()

      src_pay = iota_in if first else src_p

      def pwin(w, c):
        off = t * CH + w * W
        cp1 = pltpu.async_copy(my_pd.at[pl.ds(off, W)], posbuf, sem1)
        cp2 = pltpu.async_copy(src_pay.at[pl.ds(off, W)], paybuf, sem2)
        cp1.wait()
        cp2.wait()
        pltpu.sync_copy(paybuf, vs_sh.at[posbuf])
        return c

      lax.fori_loop(0, NW, pwin, 0)
      plsc.subcore_barrier()
    else:
      def pwin(w, c):
        off = t * CH + w * W
        cp1 = pltpu.async_copy(src_k.at[pl.ds(off, W)], keywin, sem1)
        cp2 = pltpu.async_copy(src_p.at[pl.ds(off, W)], paybuf, sem2)
        cp1.wait()
        cp2.wait()

        lax.fori_loop(0, NV // UNROLL, _rank_vregs(keywin, posbuf, offs, shift), 0)
        pltpu.sync_copy(posbuf, vs_sh.at[paybuf])
        return c

      lax.fori_loop(0, NW, pwin, 0)
      plsc.subcore_barrier()

    dst = my_rank if is_last else dst_p
    pltpu.sync_copy(vs_sh.at[pl.ds(t * CH, CH)], dst.at[pl.ds(t * CH, CH)])
    plsc.subcore_barrier()

  for p in range(NPASS):
    one_pass(p)


_buf = jax.ShapeDtypeStruct((NC, NP), jnp.int32)
_sc_rank = functools.partial(
    pl.kernel,
    out_type=(_buf, _buf, _buf, _buf, _buf, _buf),
    mesh=_mesh,
    scratch_types=[
        pltpu.VMEM_SHARED((NT * RADIX,), jnp.int32),
        pltpu.VMEM_SHARED((NP,), jnp.int32),
        pltpu.VMEM((W,), jnp.int32),
        pltpu.VMEM((W,), jnp.int32),
        pltpu.VMEM((W,), jnp.int32),
        pltpu.VMEM((NT * RADIX,), jnp.int32),
        pltpu.VMEM((RADIX,), jnp.int32),
        pltpu.VMEM((RADIX,), jnp.int32),
        pltpu.SemaphoreType.DMA,
        pltpu.SemaphoreType.DMA,
    ],
    compiler_params=pltpu.CompilerParams(needs_layout_passes=False),
)(_sc_body)


ROWS = NP // 128
BLK = 128
GRID = ROWS // BLK
MEAN = (N - 1) / 2.0


def _dot_body(rp_ref, rt_ref, acc_ref):
  i = pl.program_id(0)
  rp = rp_ref[...].astype(jnp.float32) - MEAN
  rt = rt_ref[...].astype(jnp.float32) - MEAN
  r = lax.broadcasted_iota(jnp.int32, (BLK, 128), 0)
  c = lax.broadcasted_iota(jnp.int32, (BLK, 128), 1)
  gidx = (i * BLK + r) * 128 + c
  s = jnp.sum(jnp.where(gidx < N, rp * rt, 0.0))

  @pl.when(i == 0)
  def _():
    acc_ref[0, 0] = 0.0

  acc_ref[0, 0] += s


_dot = pl.pallas_call(
    _dot_body,
    grid=(GRID,),
    in_specs=[
        pl.BlockSpec((BLK, 128), lambda i: (i, 0)),
        pl.BlockSpec((BLK, 128), lambda i: (i, 0)),
    ],
    out_specs=pl.BlockSpec(memory_space=pltpu.SMEM),
    out_shape=jax.ShapeDtypeStruct((1, 1), jnp.float32),
)


def kernel(pred, target):
  bp = lax.bitcast_convert_type(pred, jnp.int32)
  bt = lax.bitcast_convert_type(target, jnp.int32)
  sign = jnp.int32(-2147483648)
  kp = jnp.where(bp >= 0, bp ^ sign, ~bp)
  kt = jnp.where(bt >= 0, bt ^ sign, ~bt)
  pad = jnp.full((NP - N,), -1, jnp.int32)
  keys = jnp.stack([jnp.concatenate([kp, pad]), jnp.concatenate([kt, pad])])
  iota = jnp.arange(NP, dtype=jnp.int32)
  ranks = _sc_rank(keys, iota)[0]
  rp2d = ranks[0].reshape(ROWS, 128)
  rt2d = ranks[1].reshape(ROWS, 128)
  s = _dot(rp2d, rt2d)[0, 0]
  denom = float(N) * (float(N) * (N + 1) / 12.0)
  return s * jnp.float32(1.0 / denom)


# speedup vs baseline: 10.0197x; 1.1533x over previous
import functools

import jax
import jax.numpy as jnp
from jax import lax
from jax.experimental import pallas as pl
from jax.experimental.pallas import tpu as pltpu
from jax.experimental.pallas import tpu_sc as plsc

N = 1_000_000
NC = 2
NT = 16
CH = 63_488
NP = NT * CH
W = 7_936
NW = CH // W
NV = W // 16
RADIX = 256
NPASS = 4

_mesh = plsc.VectorSubcoreMesh(
    core_axis_name="c", subcore_axis_name="s", num_cores=NC, num_subcores=NT
)


def _digit(vec, shift):
  return lax.shift_right_logical(vec, jnp.full((16,), shift, jnp.int32)) & 0xFF


UNROLL = 4


def _rank_vregs(keywin, posbuf, offs, shift):
  def body(i, c):
    base = i * (16 * UNROLL)
    ds = [_digit(keywin[pl.ds(base + 16 * u, 16)], shift) for u in range(UNROLL)]
    sc = [plsc.scan_count(d) for d in ds]
    for u in range(UNROLL):
      occ, lastm = sc[u]
      cur = plsc.load_gather(offs, [ds[u]])
      posbuf[pl.ds(base + 16 * u, 16)] = cur + occ - 1
      plsc.store_scatter(offs, [ds[u]], cur + occ, mask=lastm)
    return c

  return body


HALF = NW // 2


def _sc_body(keys, iota_in, ranks, ka, pa, kb, pb, posdump,
             grid_sh, vs_sh, keyA, keyB, posA, posB, payA, payB,
             gridvm, offs, hist, sGA, sGB, sSA, sDA, sSB, sDB):
  cid = lax.axis_index("c")
  t = lax.axis_index("s")
  zeros16 = jnp.zeros((16,), jnp.int32)
  ones16 = jnp.full((16,), 1, jnp.int32)

  my_keys = keys.at[cid]
  my_rank = ranks.at[cid]
  my_pd = posdump.at[cid]
  bufs = [(ka.at[cid], pa.at[cid]), (kb.at[cid], pb.at[cid])]

  def zero_hist():
    def zbody(i, c):
      hist[pl.ds(i * 16, 16)] = zeros16
      return c

    lax.fori_loop(0, RADIX // 16, zbody, 0)

  def hist_update(d):
    plsc.addupdate_scatter(hist, [d], ones16)

  def publish_hist():
    pltpu.sync_copy(hist, grid_sh.at[pl.ds(t * RADIX, RADIX)])

  def hist_vregs(buf, shift):
    def body(i, c):
      for u in range(UNROLL):
        hist_update(_digit(buf[pl.ds(i * 16 * UNROLL + 16 * u, 16)], shift))
      return c

    return body

  def hist_phase(src, shift, dst_k):
    zero_hist()
    pltpu.async_copy(src.at[pl.ds(t * CH, W)], keyA, sGA)

    def pair(wp, c):
      off0 = t * CH + 2 * wp * W
      off1 = off0 + W
      cp_b = pltpu.async_copy(src.at[pl.ds(off1, W)], keyB, sGB)
      pltpu.make_async_copy(src.at[pl.ds(off0, W)], keyA, sGA).wait()
      lax.fori_loop(0, NV // UNROLL, hist_vregs(keyA, shift), 0)
      if dst_k is not None:
        cp_wa = pltpu.async_copy(keyA, dst_k.at[pl.ds(off0, W)], sSA)
      cp_b.wait()
      lax.fori_loop(0, NV // UNROLL, hist_vregs(keyB, shift), 0)
      if dst_k is not None:
        cp_wa.wait()

      @pl.when(wp < HALF - 1)
      def _():
        pltpu.async_copy(src.at[pl.ds(off0 + 2 * W, W)], keyA, sGA)

      if dst_k is not None:
        pltpu.async_copy(keyB, dst_k.at[pl.ds(off1, W)], sSB).wait()
      return c

    lax.fori_loop(0, HALF, pair, 0)
    publish_hist()

  hist_phase(my_keys, 0, None)
  plsc.subcore_barrier()

  def compute_offs():
    def obody(c, carry):
      def racc(tp, tb):
        tot, below = tb
        row = gridvm[pl.ds(tp * RADIX + c * 16, 16)]
        return tot + row, below + jnp.where(tp < t, row, jnp.zeros_like(row))

      tot, below = lax.fori_loop(0, NT, racc, (zeros16, zeros16))
      cs = plsc.cumsum(tot)
      offs[pl.ds(c * 16, 16)] = carry + (cs - tot) + below
      return carry + jnp.sum(tot)

    lax.fori_loop(0, RADIX // 16, obody, jnp.int32(0))

  def one_pass(p):
    shift = 8 * p
    first = p == 0
    is_last = p == NPASS - 1
    src_k = my_keys if first else bufs[(p + 1) % 2][0]
    src_p = None if first else bufs[(p + 1) % 2][1]
    dst_k, dst_p = bufs[p % 2]

    pltpu.sync_copy(grid_sh, gridvm)
    compute_offs()

    if not is_last:
      pltpu.async_copy(src_k.at[pl.ds(t * CH, W)], keyA, sGA)

      def kpair(wp, c):
        off0 = t * CH + 2 * wp * W
        off1 = off0 + W
        cp_b = pltpu.async_copy(src_k.at[pl.ds(off1, W)], keyB, sGB)
        pltpu.make_async_copy(src_k.at[pl.ds(off0, W)], keyA, sGA).wait()
        lax.fori_loop(0, NV // UNROLL, _rank_vregs(keyA, posA, offs, shift), 0)
        cp_sa = pltpu.async_copy(keyA, vs_sh.at[posA], sSA)
        cp_da = pltpu.async_copy(posA, my_pd.at[pl.ds(off0, W)], sDA)
        cp_b.wait()
        lax.fori_loop(0, NV // UNROLL, _rank_vregs(keyB, posB, offs, shift), 0)
        cp_sa.wait()
        cp_da.wait()

        @pl.when(wp < HALF - 1)
        def _():
          pltpu.async_copy(src_k.at[pl.ds(off0 + 2 * W, W)], keyA, sGA)

        cp_sb = pltpu.async_copy(keyB, vs_sh.at[posB], sSB)
        cp_db = pltpu.async_copy(posB, my_pd.at[pl.ds(off1, W)], sDB)
        cp_sb.wait()
        cp_db.wait()
        return c

      lax.fori_loop(0, HALF, kpair, 0)
      plsc.subcore_barrier()

      hist_phase(vs_sh, shift + 8, dst_k)
      plsc.subcore_barrier()

      src_pay = iota_in if first else src_p
      pltpu.async_copy(my_pd.at[pl.ds(t * CH, W)], posA, sGA)
      pltpu.async_copy(src_pay.at[pl.ds(t * CH, W)], payA, sDA)

      def ppair(wp, c):
        off0 = t * CH + 2 * wp * W
        off1 = off0 + W
        cp1 = pltpu.async_copy(my_pd.at[pl.ds(off1, W)], posB, sGB)
        cp2 = pltpu.async_copy(src_pay.at[pl.ds(off1, W)], payB, sDB)
        pltpu.make_async_copy(my_pd.at[pl.ds(off0, W)], posA, sGA).wait()
        pltpu.make_async_copy(src_pay.at[pl.ds(off0, W)], payA, sDA).wait()
        pltpu.sync_copy(payA, vs_sh.at[posA])

        @pl.when(wp < HALF - 1)
        def _():
          pltpu.async_copy(my_pd.at[pl.ds(off0 + 2 * W, W)], posA, sGA)
          pltpu.async_copy(src_pay.at[pl.ds(off0 + 2 * W, W)], payA, sDA)

        cp1.wait()
        cp2.wait()
        pltpu.sync_copy(payB, vs_sh.at[posB])
        return c

      lax.fori_loop(0, HALF, ppair, 0)
      plsc.subcore_barrier()
    else:
      pltpu.async_copy(src_k.at[pl.ds(t * CH, W)], keyA, sGA)
      pltpu.async_copy(src_p.at[pl.ds(t * CH, W)], payA, sDA)

      def rpair(wp, c):
        off0 = t * CH + 2 * wp * W
        off1 = off0 + W
        cp1 = pltpu.async_copy(src_k.at[pl.ds(off1, W)], keyB, sGB)
        cp2 = pltpu.async_copy(src_p.at[pl.ds(off1, W)], payB, sDB)
        pltpu.make_async_copy(src_k.at[pl.ds(off0, W)], keyA, sGA).wait()
        pltpu.make_async_copy(src_p.at[pl.ds(off0, W)], payA, sDA).wait()
        lax.fori_loop(0, NV // UNROLL, _rank_vregs(keyA, posA, offs, shift), 0)
        cp_sa = pltpu.async_copy(posA, vs_sh.at[payA], sSA)
        cp1.wait()
        cp2.wait()
        lax.fori_loop(0, NV // UNROLL, _rank_vregs(keyB, posB, offs, shift), 0)
        cp_sa.wait()

        @pl.when(wp < HALF - 1)
        def _():
          pltpu.async_copy(src_k.at[pl.ds(off0 + 2 * W, W)], keyA, sGA)
          pltpu.async_copy(src_p.at[pl.ds(off0 + 2 * W, W)], payA, sDA)

        pltpu.async_copy(posB, vs_sh.at[payB], sSB).wait()
        return c

      lax.fori_loop(0, HALF, rpair, 0)
      plsc.subcore_barrier()

    dst = my_rank if is_last else dst_p
    pltpu.sync_copy(vs_sh.at[pl.ds(t * CH, CH)], dst.at[pl.ds(t * CH, CH)])
    plsc.subcore_barrier()

  for p in range(NPASS):
    one_pass(p)


_buf = jax.ShapeDtypeStruct((NC, NP), jnp.int32)
_sc_rank = functools.partial(
    pl.kernel,
    out_type=(_buf, _buf, _buf, _buf, _buf, _buf),
    mesh=_mesh,
    scratch_types=[
        pltpu.VMEM_SHARED((NT * RADIX,), jnp.int32),
        pltpu.VMEM_SHARED((NP,), jnp.int32),
        pltpu.VMEM((W,), jnp.int32),
        pltpu.VMEM((W,), jnp.int32),
        pltpu.VMEM((W,), jnp.int32),
        pltpu.VMEM((W,), jnp.int32),
        pltpu.VMEM((W,), jnp.int32),
        pltpu.VMEM((W,), jnp.int32),
        pltpu.VMEM((NT * RADIX,), jnp.int32),
        pltpu.VMEM((RADIX,), jnp.int32),
        pltpu.VMEM((RADIX,), jnp.int32),
        pltpu.SemaphoreType.DMA,
        pltpu.SemaphoreType.DMA,
        pltpu.SemaphoreType.DMA,
        pltpu.SemaphoreType.DMA,
        pltpu.SemaphoreType.DMA,
        pltpu.SemaphoreType.DMA,
    ],
    compiler_params=pltpu.CompilerParams(needs_layout_passes=False),
)(_sc_body)


ROWS = NP // 128
BLK = 128
GRID = ROWS // BLK
MEAN = (N - 1) / 2.0


def _dot_body(rp_ref, rt_ref, acc_ref):
  i = pl.program_id(0)
  rp = rp_ref[...].astype(jnp.float32) - MEAN
  rt = rt_ref[...].astype(jnp.float32) - MEAN
  r = lax.broadcasted_iota(jnp.int32, (BLK, 128), 0)
  c = lax.broadcasted_iota(jnp.int32, (BLK, 128), 1)
  gidx = (i * BLK + r) * 128 + c
  s = jnp.sum(jnp.where(gidx < N, rp * rt, 0.0))

  @pl.when(i == 0)
  def _():
    acc_ref[0, 0] = 0.0

  acc_ref[0, 0] += s


_dot = pl.pallas_call(
    _dot_body,
    grid=(GRID,),
    in_specs=[
        pl.BlockSpec((BLK, 128), lambda i: (i, 0)),
        pl.BlockSpec((BLK, 128), lambda i: (i, 0)),
    ],
    out_specs=pl.BlockSpec(memory_space=pltpu.SMEM),
    out_shape=jax.ShapeDtypeStruct((1, 1), jnp.float32),
)


def kernel(pred, target):
  bp = lax.bitcast_convert_type(pred, jnp.int32)
  bt = lax.bitcast_convert_type(target, jnp.int32)
  sign = jnp.int32(-2147483648)
  kp = jnp.where(bp >= 0, bp ^ sign, ~bp)
  kt = jnp.where(bt >= 0, bt ^ sign, ~bt)
  pad = jnp.full((NP - N,), -1, jnp.int32)
  keys = jnp.stack([jnp.concatenate([kp, pad]), jnp.concatenate([kt, pad])])
  iota = jnp.arange(NP, dtype=jnp.int32)
  ranks = _sc_rank(keys, iota)[0]
  rp2d = ranks[0].reshape(ROWS, 128)
  rt2d = ranks[1].reshape(ROWS, 128)
  s = _dot(rp2d, rt2d)[0, 0]
  denom = float(N) * (float(N) * (N + 1) / 12.0)
  return s * jnp.float32(1.0 / denom)


# speedup vs baseline: 10.0611x; 1.0041x over previous
import functools

import jax
import jax.numpy as jnp
from jax import lax
from jax.experimental import pallas as pl
from jax.experimental.pallas import tpu as pltpu
from jax.experimental.pallas import tpu_sc as plsc

N = 1_000_000
NC = 2
NT = 16
CH = 63_488
NP = NT * CH
W = 7_936
NW = CH // W
NV = W // 16
RADIX = 256
NPASS = 4

_mesh = plsc.VectorSubcoreMesh(
    core_axis_name="c", subcore_axis_name="s", num_cores=NC, num_subcores=NT
)


def _digit(vec, shift):
  return lax.shift_right_logical(vec, jnp.full((16,), shift, jnp.int32)) & 0xFF


UNROLL = 4


def _rank_vregs(keywin, posbuf, offs, shift):
  def body(i, c):
    base = i * (16 * UNROLL)
    ds = [_digit(keywin[pl.ds(base + 16 * u, 16)], shift) for u in range(UNROLL)]
    sc = [plsc.scan_count(d) for d in ds]
    for u in range(UNROLL):
      occ, lastm = sc[u]
      cur = plsc.load_gather(offs, [ds[u]])
      posbuf[pl.ds(base + 16 * u, 16)] = cur + occ - 1
      plsc.store_scatter(offs, [ds[u]], cur + occ, mask=lastm)
    return c

  return body


HALF = NW // 2


def _sc_body(keys, iota_in, ranks, ka, pa, kb, pb, posdump,
             grid_sh, vs_sh, keyA, keyB, posA, posB, payA, payB,
             gridvm, offs, hist, sGA, sGB, sSA, sDA, sSB, sDB):
  cid = lax.axis_index("c")
  t = lax.axis_index("s")
  zeros16 = jnp.zeros((16,), jnp.int32)
  ones16 = jnp.full((16,), 1, jnp.int32)

  my_keys = keys.at[cid]
  my_rank = ranks.at[cid]
  my_pd = posdump.at[cid]
  bufs = [(ka.at[cid], pa.at[cid]), (kb.at[cid], pb.at[cid])]

  def zero_hist():
    def zbody(i, c):
      hist[pl.ds(i * 16, 16)] = zeros16
      return c

    lax.fori_loop(0, RADIX // 16, zbody, 0)

  def hist_update(d):
    plsc.addupdate_scatter(hist, [d], ones16)

  def publish_hist():
    pltpu.sync_copy(hist, grid_sh.at[pl.ds(t * RADIX, RADIX)])

  def hist_vregs(buf, shift):
    hu = 2 * UNROLL

    def body(i, c):
      for u in range(hu):
        hist_update(_digit(buf[pl.ds(i * 16 * hu + 16 * u, 16)], shift))
      return c

    return body

  def hist_phase(src, shift, dst_k):
    zero_hist()
    pltpu.async_copy(src.at[pl.ds(t * CH, W)], keyA, sGA)

    def pair(wp, c):
      off0 = t * CH + 2 * wp * W
      off1 = off0 + W
      cp_b = pltpu.async_copy(src.at[pl.ds(off1, W)], keyB, sGB)
      pltpu.make_async_copy(src.at[pl.ds(off0, W)], keyA, sGA).wait()
      lax.fori_loop(0, NV // (2 * UNROLL), hist_vregs(keyA, shift), 0)
      if dst_k is not None:
        cp_wa = pltpu.async_copy(keyA, dst_k.at[pl.ds(off0, W)], sSA)
      cp_b.wait()
      lax.fori_loop(0, NV // (2 * UNROLL), hist_vregs(keyB, shift), 0)
      if dst_k is not None:
        cp_wa.wait()

      @pl.when(wp < HALF - 1)
      def _():
        pltpu.async_copy(src.at[pl.ds(off0 + 2 * W, W)], keyA, sGA)

      if dst_k is not None:
        pltpu.async_copy(keyB, dst_k.at[pl.ds(off1, W)], sSB).wait()
      return c

    lax.fori_loop(0, HALF, pair, 0)
    publish_hist()

  hist_phase(my_keys, 0, None)
  plsc.subcore_barrier()

  def compute_offs():
    def obody(c, carry):
      def racc(tp, tb):
        tot, below = tb
        row = gridvm[pl.ds(tp * RADIX + c * 16, 16)]
        return tot + row, below + jnp.where(tp < t, row, jnp.zeros_like(row))

      tot, below = lax.fori_loop(0, NT, racc, (zeros16, zeros16))
      cs = plsc.cumsum(tot)
      offs[pl.ds(c * 16, 16)] = carry + (cs - tot) + below
      return carry + jnp.sum(tot)

    lax.fori_loop(0, RADIX // 16, obody, jnp.int32(0))

  def one_pass(p):
    shift = 8 * p
    first = p == 0
    is_last = p == NPASS - 1
    src_k = my_keys if first else bufs[(p + 1) % 2][0]
    src_p = None if first else bufs[(p + 1) % 2][1]
    dst_k, dst_p = bufs[p % 2]

    pltpu.sync_copy(grid_sh, gridvm)
    compute_offs()

    if not is_last:
      pltpu.async_copy(src_k.at[pl.ds(t * CH, W)], keyA, sGA)

      def kpair(wp, c):
        off0 = t * CH + 2 * wp * W
        off1 = off0 + W
        cp_b = pltpu.async_copy(src_k.at[pl.ds(off1, W)], keyB, sGB)
        pltpu.make_async_copy(src_k.at[pl.ds(off0, W)], keyA, sGA).wait()
        lax.fori_loop(0, NV // UNROLL, _rank_vregs(keyA, posA, offs, shift), 0)
        cp_sa = pltpu.async_copy(keyA, vs_sh.at[posA], sSA)
        cp_da = pltpu.async_copy(posA, my_pd.at[pl.ds(off0, W)], sDA)
        cp_b.wait()
        lax.fori_loop(0, NV // UNROLL, _rank_vregs(keyB, posB, offs, shift), 0)
        cp_sa.wait()
        cp_da.wait()

        @pl.when(wp < HALF - 1)
        def _():
          pltpu.async_copy(src_k.at[pl.ds(off0 + 2 * W, W)], keyA, sGA)

        cp_sb = pltpu.async_copy(keyB, vs_sh.at[posB], sSB)
        cp_db = pltpu.async_copy(posB, my_pd.at[pl.ds(off1, W)], sDB)
        cp_sb.wait()
        cp_db.wait()
        return c

      lax.fori_loop(0, HALF, kpair, 0)
      plsc.subcore_barrier()

      hist_phase(vs_sh, shift + 8, dst_k)
      plsc.subcore_barrier()

      src_pay = iota_in if first else src_p
      pltpu.async_copy(my_pd.at[pl.ds(t * CH, W)], posA, sGA)
      pltpu.async_copy(src_pay.at[pl.ds(t * CH, W)], payA, sDA)

      def ppair(wp, c):
        off0 = t * CH + 2 * wp * W
        off1 = off0 + W
        cp1 = pltpu.async_copy(my_pd.at[pl.ds(off1, W)], posB, sGB)
        cp2 = pltpu.async_copy(src_pay.at[pl.ds(off1, W)], payB, sDB)
        pltpu.make_async_copy(my_pd.at[pl.ds(off0, W)], posA, sGA).wait()
        pltpu.make_async_copy(src_pay.at[pl.ds(off0, W)], payA, sDA).wait()
        pltpu.sync_copy(payA, vs_sh.at[posA])

        @pl.when(wp < HALF - 1)
        def _():
          pltpu.async_copy(my_pd.at[pl.ds(off0 + 2 * W, W)], posA, sGA)
          pltpu.async_copy(src_pay.at[pl.ds(off0 + 2 * W, W)], payA, sDA)

        cp1.wait()
        cp2.wait()
        pltpu.sync_copy(payB, vs_sh.at[posB])
        return c

      lax.fori_loop(0, HALF, ppair, 0)
      plsc.subcore_barrier()
    else:
      pltpu.async_copy(src_k.at[pl.ds(t * CH, W)], keyA, sGA)
      pltpu.async_copy(src_p.at[pl.ds(t * CH, W)], payA, sDA)

      def rpair(wp, c):
        off0 = t * CH + 2 * wp * W
        off1 = off0 + W
        cp1 = pltpu.async_copy(src_k.at[pl.ds(off1, W)], keyB, sGB)
        cp2 = pltpu.async_copy(src_p.at[pl.ds(off1, W)], payB, sDB)
        pltpu.make_async_copy(src_k.at[pl.ds(off0, W)], keyA, sGA).wait()
        pltpu.make_async_copy(src_p.at[pl.ds(off0, W)], payA, sDA).wait()
        lax.fori_loop(0, NV // UNROLL, _rank_vregs(keyA, posA, offs, shift), 0)
        cp_sa = pltpu.async_copy(posA, vs_sh.at[payA], sSA)
        cp1.wait()
        cp2.wait()
        lax.fori_loop(0, NV // UNROLL, _rank_vregs(keyB, posB, offs, shift), 0)
        cp_sa.wait()

        @pl.when(wp < HALF - 1)
        def _():
          pltpu.async_copy(src_k.at[pl.ds(off0 + 2 * W, W)], keyA, sGA)
          pltpu.async_copy(src_p.at[pl.ds(off0 + 2 * W, W)], payA, sDA)

        pltpu.async_copy(posB, vs_sh.at[payB], sSB).wait()
        return c

      lax.fori_loop(0, HALF, rpair, 0)
      plsc.subcore_barrier()

    dst = my_rank if is_last else dst_p
    pltpu.sync_copy(vs_sh.at[pl.ds(t * CH, CH)], dst.at[pl.ds(t * CH, CH)])
    plsc.subcore_barrier()

  for p in range(NPASS):
    one_pass(p)


_buf = jax.ShapeDtypeStruct((NC, NP), jnp.int32)
_sc_rank = functools.partial(
    pl.kernel,
    out_type=(_buf, _buf, _buf, _buf, _buf, _buf),
    mesh=_mesh,
    scratch_types=[
        pltpu.VMEM_SHARED((NT * RADIX,), jnp.int32),
        pltpu.VMEM_SHARED((NP,), jnp.int32),
        pltpu.VMEM((W,), jnp.int32),
        pltpu.VMEM((W,), jnp.int32),
        pltpu.VMEM((W,), jnp.int32),
        pltpu.VMEM((W,), jnp.int32),
        pltpu.VMEM((W,), jnp.int32),
        pltpu.VMEM((W,), jnp.int32),
        pltpu.VMEM((NT * RADIX,), jnp.int32),
        pltpu.VMEM((RADIX,), jnp.int32),
        pltpu.VMEM((RADIX,), jnp.int32),
        pltpu.SemaphoreType.DMA,
        pltpu.SemaphoreType.DMA,
        pltpu.SemaphoreType.DMA,
        pltpu.SemaphoreType.DMA,
        pltpu.SemaphoreType.DMA,
        pltpu.SemaphoreType.DMA,
    ],
    compiler_params=pltpu.CompilerParams(needs_layout_passes=False),
)(_sc_body)


ROWS = NP // 128
BLK = 128
GRID = ROWS // BLK
MEAN = (N - 1) / 2.0


def _dot_body(rp_ref, rt_ref, acc_ref):
  i = pl.program_id(0)
  rp = rp_ref[...].astype(jnp.float32) - MEAN
  rt = rt_ref[...].astype(jnp.float32) - MEAN
  r = lax.broadcasted_iota(jnp.int32, (BLK, 128), 0)
  c = lax.broadcasted_iota(jnp.int32, (BLK, 128), 1)
  gidx = (i * BLK + r) * 128 + c
  s = jnp.sum(jnp.where(gidx < N, rp * rt, 0.0))

  @pl.when(i == 0)
  def _():
    acc_ref[0, 0] = 0.0

  acc_ref[0, 0] += s


_dot = pl.pallas_call(
    _dot_body,
    grid=(GRID,),
    in_specs=[
        pl.BlockSpec((BLK, 128), lambda i: (i, 0)),
        pl.BlockSpec((BLK, 128), lambda i: (i, 0)),
    ],
    out_specs=pl.BlockSpec(memory_space=pltpu.SMEM),
    out_shape=jax.ShapeDtypeStruct((1, 1), jnp.float32),
)


def kernel(pred, target):
  bp = lax.bitcast_convert_type(pred, jnp.int32)
  bt = lax.bitcast_convert_type(target, jnp.int32)
  sign = jnp.int32(-2147483648)
  kp = jnp.where(bp >= 0, bp ^ sign, ~bp)
  kt = jnp.where(bt >= 0, bt ^ sign, ~bt)
  pad = jnp.full((NP - N,), -1, jnp.int32)
  keys = jnp.stack([jnp.concatenate([kp, pad]), jnp.concatenate([kt, pad])])
  iota = jnp.arange(NP, dtype=jnp.int32)
  ranks = _sc_rank(keys, iota)[0]
  rp2d = ranks[0].reshape(ROWS, 128)
  rt2d = ranks[1].reshape(ROWS, 128)
  s = _dot(rp2d, rt2d)[0, 0]
  denom = float(N) * (float(N) * (N + 1) / 12.0)
  return s * jnp.float32(1.0 / denom)
